# baseline (speedup 1.0000x reference)
import jax
import jax.numpy as jnp
from jax.experimental import pallas as pl
from jax.experimental.pallas import tpu as pltpu

_B, _S, _D = 4, 8192, 1024
_CS = 512
_NSP = _S // _CS
_NXB = 4
_LOOK = 2


def _body(x_hbm, pos_hbm, out_hbm, xbufs, pbufs, load_sems, store_sems,
          pos_sems):
    nt = _NSP * _B
    loads = [None] * nt
    stores = [None] * nt
    pos_loads = [None] * _NSP

    def start_load(t):
        k = t % _NXB
        sp, b = divmod(t, _B)
        loads[t] = pltpu.make_async_copy(
            x_hbm.at[b, pl.ds(sp * _CS, _CS)], xbufs.at[k], load_sems.at[k])
        loads[t].start()

    def start_pos_load(sp):
        pos_loads[sp] = pltpu.make_async_copy(
            pos_hbm.at[pl.ds(sp * _CS, _CS)], pbufs.at[sp % 2],
            pos_sems.at[sp % 2])
        pos_loads[sp].start()

    start_pos_load(0)
    start_pos_load(1)
    for t in range(_LOOK):
        start_load(t)

    for t in range(nt):
        k = t % _NXB
        sp, b = divmod(t, _B)
        if b == 0:
            pos_loads[sp].wait()
        loads[t].wait()
        xbufs[k] = xbufs[k] + pbufs[sp % 2]
        if b == _B - 1 and sp + 2 < _NSP:
            start_pos_load(sp + 2)
        stores[t] = pltpu.make_async_copy(
            xbufs.at[k], out_hbm.at[b, pl.ds(sp * _CS, _CS)], store_sems.at[k])
        stores[t].start()
        nxt = t + _LOOK
        if nxt < nt:
            prev = nxt - _NXB
            if prev >= 0:
                stores[prev].wait()
            start_load(nxt)

    for t in range(nt - _NXB, nt):
        stores[t].wait()


def kernel(x, pos_table):
    return pl.pallas_call(
        _body,
        in_specs=[
            pl.BlockSpec(memory_space=pl.ANY),
            pl.BlockSpec(memory_space=pl.ANY),
        ],
        out_specs=pl.BlockSpec(memory_space=pl.ANY),
        out_shape=jax.ShapeDtypeStruct((_B, _S, _D), x.dtype),
        scratch_shapes=[
            pltpu.VMEM((_NXB, _CS, _D), jnp.float32),
            pltpu.VMEM((2, _CS, _D), jnp.float32),
            pltpu.SemaphoreType.DMA((_NXB,)),
            pltpu.SemaphoreType.DMA((_NXB,)),
            pltpu.SemaphoreType.DMA((2,)),
        ],
    )(x, pos_table)


# speedup vs baseline: 1.1243x; 1.1243x over previous
import jax
import jax.numpy as jnp
from jax.experimental import pallas as pl
from jax.experimental.pallas import tpu as pltpu

_B, _S, _D = 4, 8192, 1024
_CS = 512
_NSP = _S // _CS
_NXB = 6
_LOOK = 3


def _body(x_hbm, pos_hbm, out_hbm, xbufs, pbufs, load_sems, store_sems,
          pos_sems):
    nt = _NSP * _B
    loads = [None] * nt
    stores = [None] * nt
    pos_loads = [None] * _NSP

    def start_load(t):
        k = t % _NXB
        sp, b = divmod(t, _B)
        loads[t] = pltpu.make_async_copy(
            x_hbm.at[b, pl.ds(sp * _CS, _CS)], xbufs.at[k], load_sems.at[k])
        loads[t].start()

    def start_pos_load(sp):
        pos_loads[sp] = pltpu.make_async_copy(
            pos_hbm.at[pl.ds(sp * _CS, _CS)], pbufs.at[sp % 2],
            pos_sems.at[sp % 2])
        pos_loads[sp].start()

    start_pos_load(0)
    start_pos_load(1)
    for t in range(_LOOK):
        start_load(t)

    for t in range(nt):
        k = t % _NXB
        sp, b = divmod(t, _B)
        if b == 0:
            pos_loads[sp].wait()
        loads[t].wait()
        xbufs[k] = xbufs[k] + pbufs[sp % 2]
        if b == _B - 1 and sp + 2 < _NSP:
            start_pos_load(sp + 2)
        stores[t] = pltpu.make_async_copy(
            xbufs.at[k], out_hbm.at[b, pl.ds(sp * _CS, _CS)], store_sems.at[k])
        stores[t].start()
        nxt = t + _LOOK
        if nxt < nt:
            prev = nxt - _NXB
            if prev >= 0:
                stores[prev].wait()
            start_load(nxt)

    for t in range(nt - _NXB, nt):
        stores[t].wait()


def kernel(x, pos_table):
    return pl.pallas_call(
        _body,
        in_specs=[
            pl.BlockSpec(memory_space=pl.ANY),
            pl.BlockSpec(memory_space=pl.ANY),
        ],
        out_specs=pl.BlockSpec(memory_space=pl.ANY),
        out_shape=jax.ShapeDtypeStruct((_B, _S, _D), x.dtype),
        scratch_shapes=[
            pltpu.VMEM((_NXB, _CS, _D), jnp.float32),
            pltpu.VMEM((2, _CS, _D), jnp.float32),
            pltpu.SemaphoreType.DMA((_NXB,)),
            pltpu.SemaphoreType.DMA((_NXB,)),
            pltpu.SemaphoreType.DMA((2,)),
        ],
    )(x, pos_table)


# speedup vs baseline: 1.1374x; 1.0117x over previous
import jax
import jax.numpy as jnp
from jax.experimental import pallas as pl
from jax.experimental.pallas import tpu as pltpu

_B, _S, _D = 4, 8192, 1024
_CS = 512
_NSP = _S // _CS
_NXB = 5
_LOOK = 4
_NOB = 6


def _body(x_hbm, pos_hbm, out_hbm, xbufs, obufs, pbufs, load_sems,
          store_sems, pos_sems):
    nt = _NSP * _B
    loads = [None] * nt
    stores = [None] * nt
    pos_loads = [None] * _NSP

    def start_load(t):
        k = t % _NXB
        sp, b = divmod(t, _B)
        loads[t] = pltpu.make_async_copy(
            x_hbm.at[b, pl.ds(sp * _CS, _CS)], xbufs.at[k], load_sems.at[k])
        loads[t].start()

    def start_pos_load(sp):
        pos_loads[sp] = pltpu.make_async_copy(
            pos_hbm.at[pl.ds(sp * _CS, _CS)], pbufs.at[sp % 2],
            pos_sems.at[sp % 2])
        pos_loads[sp].start()

    start_pos_load(0)
    start_pos_load(1)
    for t in range(_LOOK):
        start_load(t)

    for t in range(nt):
        kx = t % _NXB
        ko = t % _NOB
        sp, b = divmod(t, _B)
        if b == 0:
            pos_loads[sp].wait()
        loads[t].wait()
        if t - _NOB >= 0:
            stores[t - _NOB].wait()
        obufs[ko] = xbufs[kx] + pbufs[sp % 2]
        if b == _B - 1 and sp + 2 < _NSP:
            start_pos_load(sp + 2)
        stores[t] = pltpu.make_async_copy(
            obufs.at[ko], out_hbm.at[b, pl.ds(sp * _CS, _CS)],
            store_sems.at[ko])
        stores[t].start()
        if t + _LOOK < nt:
            start_load(t + _LOOK)

    for t in range(nt - _NOB, nt):
        stores[t].wait()


def kernel(x, pos_table):
    return pl.pallas_call(
        _body,
        in_specs=[
            pl.BlockSpec(memory_space=pl.ANY),
            pl.BlockSpec(memory_space=pl.ANY),
        ],
        out_specs=pl.BlockSpec(memory_space=pl.ANY),
        out_shape=jax.ShapeDtypeStruct((_B, _S, _D), x.dtype),
        scratch_shapes=[
            pltpu.VMEM((_NXB, _CS, _D), jnp.float32),
            pltpu.VMEM((_NOB, _CS, _D), jnp.float32),
            pltpu.VMEM((2, _CS, _D), jnp.float32),
            pltpu.SemaphoreType.DMA((_NXB,)),
            pltpu.SemaphoreType.DMA((_NOB,)),
            pltpu.SemaphoreType.DMA((2,)),
        ],
    )(x, pos_table)


# speedup vs baseline: 1.1377x; 1.0003x over previous
import jax
import jax.numpy as jnp
from jax.experimental import pallas as pl

_BS = 2048


def _add_body(x_ref, pos_ref, o_ref):
    o_ref[...] = x_ref[...] + pos_ref[...][None, :, :]


def kernel(x, pos_table):
    B, S, D = x.shape
    n_s = S // _BS
    return pl.pallas_call(
        _add_body,
        grid=(n_s, B),
        in_specs=[
            pl.BlockSpec((1, _BS, D), lambda s, b: (b, s, 0)),
            pl.BlockSpec((_BS, D), lambda s, b: (s, 0)),
        ],
        out_specs=pl.BlockSpec((1, _BS, D), lambda s, b: (b, s, 0)),
        out_shape=jax.ShapeDtypeStruct((B, S, D), x.dtype),
    )(x, pos_table)
